# baseline (speedup 1.0000x reference)
import functools

import jax
import jax.numpy as jnp
from jax import lax
from jax.experimental import pallas as pl
from jax.experimental.pallas import tpu as pltpu
from jax.experimental.pallas import tpu_sc as plsc

N_NODES = 10000
N_EDGES = 320000
D = 128

NC = 2
NS = 16
NW = NC * NS

CH = 64
NBUF = 3
NCH = 168
EPT = CH * NCH
E_PAD = EPT * NW


def _normalize_body(x_ref, d_ref, s_ref, ew_ref, e_ref):
    x = x_ref[...]
    norm = jnp.sqrt(jnp.sum(x * x, axis=1, keepdims=True))
    e = x / jnp.maximum(norm, 1e-12)
    e_ref[...] = e
    ew_ref[...] = e * (d_ref[...] * s_ref[0, 0])


def _make_tables(emb, d2, s2):
    return pl.pallas_call(
        _normalize_body,
        out_shape=(
            jax.ShapeDtypeStruct((N_NODES, D), jnp.float32),
            jax.ShapeDtypeStruct((N_NODES, D), jnp.float32),
        ),
    )(emb, d2, s2)


def _sc_body(ew_hbm, e_hbm, src_hbm, dst_hbm, out_hbm,
             sidx, didx,
             srows0, srows1, srows2,
             drows0, drows1, drows2,
             outv, sem0, sem1, sem2):
    cid = lax.axis_index("c")
    sid = lax.axis_index("s")
    wid = sid * NC + cid
    srows = (srows0, srows1, srows2)
    drows = (drows0, drows1, drows2)
    sems = (sem0, sem1, sem2)

    pltpu.sync_copy(src_hbm.at[pl.ds(wid * NCH, NCH)], sidx)
    pltpu.sync_copy(dst_hbm.at[pl.ds(wid * NCH, NCH)], didx)

    def fire(j, b):
        pltpu.async_copy(ew_hbm.at[sidx.at[j]], srows[b], sems[b])
        pltpu.async_copy(e_hbm.at[didx.at[j]], drows[b], sems[b])

    def drain(b):
        pltpu.make_async_copy(ew_hbm.at[sidx.at[0]], srows[b], sems[b]).wait()
        pltpu.make_async_copy(e_hbm.at[didx.at[0]], drows[b], sems[b]).wait()

    for b in range(NBUF - 1):
        fire(b, b)

    def outer(t, _):
        for b in range(NBUF):
            j = t * NBUF + b

            @pl.when(j < NCH - (NBUF - 1))
            def _():
                fire(j + NBUF - 1, (b + NBUF - 1) % NBUF)

            drain(b)

            def group_body(g, _):
                base = g * 16
                lane = lax.iota(jnp.int32, 16)
                res = jnp.zeros((16,), jnp.float32)
                for jj in range(16):
                    i = base + jj
                    acc = jnp.zeros((16,), jnp.float32)
                    for c in range(D // 16):
                        sl = pl.ds(c * 16, 16)
                        acc = acc + srows[b][i, sl] * drows[b][i, sl]
                    dot = jnp.sum(acc)
                    res = jnp.where(lane == jj, dot, res)
                outv[pl.ds(j * CH + base, 16)] = res
                return 0

            lax.fori_loop(0, CH // 16, group_body, 0)
        return 0

    lax.fori_loop(0, NCH // NBUF, outer, 0)
    pltpu.sync_copy(outv, out_hbm.at[pl.ds(wid * EPT, EPT)])


_sc_dot = functools.partial(
    pl.kernel,
    out_type=jax.ShapeDtypeStruct((E_PAD,), jnp.float32),
    mesh=plsc.VectorSubcoreMesh(
        core_axis_name="c", subcore_axis_name="s", num_cores=NC, num_subcores=NS
    ),
    scratch_types=(
        [pltpu.VMEM((NCH, CH), jnp.int32)] * 2
        + [pltpu.VMEM((CH, D), jnp.float32)] * (2 * NBUF)
        + [pltpu.VMEM((EPT,), jnp.float32)]
        + [pltpu.SemaphoreType.DMA] * NBUF
    ),
    compiler_params=pltpu.CompilerParams(needs_layout_passes=False),
)(_sc_body)


def kernel(emb, edge_index, d, scale):
    d2 = d.astype(jnp.float32).reshape(1, D)
    s2 = scale.astype(jnp.float32).reshape(1, 1)
    ew, e = _make_tables(emb, d2, s2)
    ei = edge_index.astype(jnp.int32)
    pad = jnp.zeros((2, E_PAD - N_EDGES), jnp.int32)
    ei = jnp.concatenate([ei, pad], axis=1)
    pair = _sc_dot(ew, e, ei[0].reshape(NW * NCH, CH), ei[1].reshape(NW * NCH, CH))
    return pair[:N_EDGES].reshape(N_EDGES, 1)


# speedup vs baseline: 1.2612x; 1.2612x over previous
import functools

import jax
import jax.numpy as jnp
from jax import lax
from jax.experimental import pallas as pl
from jax.experimental.pallas import tpu as pltpu
from jax.experimental.pallas import tpu_sc as plsc

N_NODES = 10000
N_EDGES = 320000
D = 128

NC = 2
NS = 16
NW = NC * NS

CH = 256
KR = CH // 128
NCH = 40
EPT = CH * NCH
E_PAD = EPT * NW


def _normalize_body(x_ref, d_ref, s_ref, ew_ref, e_ref):
    x = x_ref[...]
    norm = jnp.sqrt(jnp.sum(x * x, axis=1, keepdims=True))
    e = x / jnp.maximum(norm, 1e-12)
    e_ref[...] = e
    ew_ref[...] = e * (d_ref[...] * s_ref[0, 0])


def _make_tables(emb, d2, s2):
    return pl.pallas_call(
        _normalize_body,
        out_shape=(
            jax.ShapeDtypeStruct((N_NODES, D), jnp.float32),
            jax.ShapeDtypeStruct((N_NODES, D), jnp.float32),
        ),
    )(emb, d2, s2)


def _sc_body(ew_hbm, e_hbm, src_hbm, dst_hbm, out_hbm,
             sidx, didx, srows, drows, outv, sem):
    cid = lax.axis_index("c")
    sid = lax.axis_index("s")
    wid = sid * NC + cid

    pltpu.sync_copy(src_hbm.at[pl.ds(wid * NCH * KR, NCH * KR)], sidx)
    pltpu.sync_copy(dst_hbm.at[pl.ds(wid * NCH * KR, NCH * KR)], didx)

    def chunk_body(j, _):
        copies = []
        for k in range(KR):
            sl = pl.ds(k * 128, 128)
            copies.append(
                pltpu.async_copy(ew_hbm.at[sidx.at[j * KR + k]], srows.at[sl], sem))
            copies.append(
                pltpu.async_copy(e_hbm.at[didx.at[j * KR + k]], drows.at[sl], sem))
        for cp in copies:
            cp.wait()

        def group_body(g, _):
            base = g * 16
            lane = lax.iota(jnp.int32, 16)
            res = jnp.zeros((16,), jnp.float32)
            for jj in range(16):
                i = base + jj
                acc = jnp.zeros((16,), jnp.float32)
                for c in range(D // 16):
                    sl = pl.ds(c * 16, 16)
                    acc = acc + srows[i, sl] * drows[i, sl]
                dot = jnp.sum(acc)
                res = jnp.where(lane == jj, dot, res)
            outv[pl.ds(j * CH + base, 16)] = res
            return 0

        lax.fori_loop(0, CH // 16, group_body, 0)
        return 0

    lax.fori_loop(0, NCH, chunk_body, 0)
    pltpu.sync_copy(outv, out_hbm.at[pl.ds(wid * EPT, EPT)])


_sc_dot = functools.partial(
    pl.kernel,
    out_type=jax.ShapeDtypeStruct((E_PAD,), jnp.float32),
    mesh=plsc.VectorSubcoreMesh(
        core_axis_name="c", subcore_axis_name="s", num_cores=NC, num_subcores=NS
    ),
    scratch_types=(
        [pltpu.VMEM((NCH * KR, 128), jnp.int32)] * 2
        + [pltpu.VMEM((CH, D), jnp.float32)] * 2
        + [pltpu.VMEM((EPT,), jnp.float32)]
        + [pltpu.SemaphoreType.DMA]
    ),
    compiler_params=pltpu.CompilerParams(needs_layout_passes=False),
)(_sc_body)


def kernel(emb, edge_index, d, scale):
    d2 = d.astype(jnp.float32).reshape(1, D)
    s2 = scale.astype(jnp.float32).reshape(1, 1)
    ew, e = _make_tables(emb, d2, s2)
    ei = edge_index.astype(jnp.int32)
    pad = jnp.zeros((2, E_PAD - N_EDGES), jnp.int32)
    ei = jnp.concatenate([ei, pad], axis=1)
    pair = _sc_dot(ew, e, ei[0].reshape(-1, 128), ei[1].reshape(-1, 128))
    return pair[:N_EDGES].reshape(N_EDGES, 1)


# speedup vs baseline: 1.2691x; 1.0063x over previous
import functools

import jax
import jax.numpy as jnp
from jax import lax
from jax.experimental import pallas as pl
from jax.experimental.pallas import tpu as pltpu
from jax.experimental.pallas import tpu_sc as plsc

N_NODES = 10000
N_EDGES = 320000
D = 128

NC = 2
NS = 16
NW = NC * NS

CH = 256
KR = CH // 128
NCH = 40
EPT = CH * NCH
E_PAD = EPT * NW


def _normalize_body(x_ref, d_ref, s_ref, ew_ref, e_ref):
    x = x_ref[...]
    norm = jnp.sqrt(jnp.sum(x * x, axis=1, keepdims=True))
    e = x / jnp.maximum(norm, 1e-12)
    e_ref[...] = e
    ew_ref[...] = e * (d_ref[...] * s_ref[0, 0])


def _make_tables(emb, d2, s2):
    return pl.pallas_call(
        _normalize_body,
        out_shape=(
            jax.ShapeDtypeStruct((N_NODES, D), jnp.float32),
            jax.ShapeDtypeStruct((N_NODES, D), jnp.float32),
        ),
    )(emb, d2, s2)


def _sc_body(ew_hbm, e_hbm, src_hbm, dst_hbm, out_hbm,
             sidx, didx, srows, drows, outv, sem):
    cid = lax.axis_index("c")
    sid = lax.axis_index("s")
    wid = sid * NC + cid

    pltpu.sync_copy(src_hbm.at[pl.ds(wid * NCH, NCH)], sidx)
    pltpu.sync_copy(dst_hbm.at[pl.ds(wid * NCH, NCH)], didx)

    def chunk_body(j, _):
        copies = []
        for k in range(KR):
            sl = pl.ds(k * 128, 128)
            copies.append(
                pltpu.async_copy(ew_hbm.at[sidx.at[j, k]], srows.at[sl], sem))
            copies.append(
                pltpu.async_copy(e_hbm.at[didx.at[j, k]], drows.at[sl], sem))
        for cp in copies:
            cp.wait()

        def group_body(g, _):
            base = g * 16
            lane = lax.iota(jnp.int32, 16)
            res = jnp.zeros((16,), jnp.float32)
            for jj in range(16):
                i = base + jj
                acc = jnp.zeros((16,), jnp.float32)
                for c in range(D // 16):
                    sl = pl.ds(c * 16, 16)
                    acc = acc + srows[i, sl] * drows[i, sl]
                dot = jnp.sum(acc)
                res = jnp.where(lane == jj, dot, res)
            outv[pl.ds(j * CH + base, 16)] = res
            return 0

        lax.fori_loop(0, CH // 16, group_body, 0)
        return 0

    lax.fori_loop(0, NCH, chunk_body, 0)
    pltpu.sync_copy(outv, out_hbm.at[pl.ds(wid * EPT, EPT)])


_sc_dot = functools.partial(
    pl.kernel,
    out_type=jax.ShapeDtypeStruct((E_PAD,), jnp.float32),
    mesh=plsc.VectorSubcoreMesh(
        core_axis_name="c", subcore_axis_name="s", num_cores=NC, num_subcores=NS
    ),
    scratch_types=(
        [pltpu.VMEM((NCH, KR, 128), jnp.int32)] * 2
        + [pltpu.VMEM((CH, D), jnp.float32)] * 2
        + [pltpu.VMEM((EPT,), jnp.float32)]
        + [pltpu.SemaphoreType.DMA]
    ),
    compiler_params=pltpu.CompilerParams(needs_layout_passes=False),
)(_sc_body)


def kernel(emb, edge_index, d, scale):
    d2 = d.astype(jnp.float32).reshape(1, D)
    s2 = scale.astype(jnp.float32).reshape(1, 1)
    ew, e = _make_tables(emb, d2, s2)
    ei = edge_index.astype(jnp.int32)
    pad = jnp.zeros((2, E_PAD - N_EDGES), jnp.int32)
    ei = jnp.concatenate([ei, pad], axis=1)
    pair = _sc_dot(ew, e, ei[0].reshape(-1, KR, 128), ei[1].reshape(-1, KR, 128))
    return pair[:N_EDGES].reshape(N_EDGES, 1)


# speedup vs baseline: 2.5442x; 2.0048x over previous
import functools

import jax
import jax.numpy as jnp
from jax import lax
from jax.experimental import pallas as pl
from jax.experimental.pallas import tpu as pltpu
from jax.experimental.pallas import tpu_sc as plsc

N_NODES = 10000
N_EDGES = 320000
D = 128

NC = 2
NS = 16
NW = NC * NS

CH = 256
KROWS = CH // 128
N_CHUNKS = N_EDGES // CH


def _normalize_body(x_ref, d_ref, s_ref, ew_ref, e_ref):
    x = x_ref[...]
    norm = jnp.sqrt(jnp.sum(x * x, axis=1, keepdims=True))
    e = x / jnp.maximum(norm, 1e-12)
    e_ref[...] = e
    ew_ref[...] = e * (d_ref[...] * s_ref[0, 0])


def _make_tables(emb, d2, s2):
    return pl.pallas_call(
        _normalize_body,
        out_shape=(
            jax.ShapeDtypeStruct((N_NODES, D), jnp.float32),
            jax.ShapeDtypeStruct((N_NODES, D), jnp.float32),
        ),
    )(emb, d2, s2)


def _sc_body(ew_hbm, e_hbm, src_hbm, dst_hbm, out_hbm,
             sidx, didx, srows, drows, outv, sem):
    wid = lax.axis_index("s") * NC + lax.axis_index("c")
    n_my = (N_CHUNKS - wid - 1) // NW + 1

    def chunk_body(j, _):
        c = wid + j * NW
        pltpu.sync_copy(src_hbm.at[c], sidx)
        pltpu.sync_copy(dst_hbm.at[c], didx)
        copies = []
        for k in range(KROWS):
            sl = pl.ds(k * 128, 128)
            copies.append(pltpu.async_copy(ew_hbm.at[sidx.at[k]], srows.at[sl], sem))
            copies.append(pltpu.async_copy(e_hbm.at[didx.at[k]], drows.at[sl], sem))
        for cp in copies:
            cp.wait()

        def group_body(g, _):
            base = g * 16
            lane = lax.iota(jnp.int32, 16)
            res = jnp.zeros((16,), jnp.float32)
            for jj in range(16):
                i = base + jj
                acc = jnp.zeros((16,), jnp.float32)
                for c2 in range(D // 16):
                    sl = pl.ds(c2 * 16, 16)
                    acc = acc + srows[i, sl] * drows[i, sl]
                dot = jnp.sum(acc)
                res = jnp.where(lane == jj, dot, res)
            outv[pl.ds(base, 16)] = res
            return 0

        lax.fori_loop(0, CH // 16, group_body, 0)
        pltpu.sync_copy(outv, out_hbm.at[pl.ds(c * CH, CH)])
        return 0

    lax.fori_loop(0, n_my, chunk_body, 0)


_sc_dot = functools.partial(
    pl.kernel,
    out_type=jax.ShapeDtypeStruct((N_EDGES,), jnp.float32),
    mesh=plsc.VectorSubcoreMesh(
        core_axis_name="c", subcore_axis_name="s", num_cores=NC, num_subcores=NS
    ),
    scratch_types=[
        pltpu.VMEM((KROWS, 128), jnp.int32),
        pltpu.VMEM((KROWS, 128), jnp.int32),
        pltpu.VMEM((CH, D), jnp.float32),
        pltpu.VMEM((CH, D), jnp.float32),
        pltpu.VMEM((CH,), jnp.float32),
        pltpu.SemaphoreType.DMA,
    ],
    compiler_params=pltpu.CompilerParams(needs_layout_passes=False),
)(_sc_body)


def kernel(emb, edge_index, d, scale):
    d2 = d.astype(jnp.float32).reshape(1, D)
    s2 = scale.astype(jnp.float32).reshape(1, 1)
    ew, e = _make_tables(emb, d2, s2)
    src = edge_index[0].astype(jnp.int32).reshape(N_CHUNKS, KROWS, 128)
    dst = edge_index[1].astype(jnp.int32).reshape(N_CHUNKS, KROWS, 128)
    pair = _sc_dot(ew, e, src, dst)
    return pair.reshape(N_EDGES, 1)
